# Initial kernel scaffold; baseline (speedup 1.0000x reference)
import functools

import jax
import jax.numpy as jnp
from jax import lax
from jax.experimental import pallas as pl
from jax.experimental.pallas import tpu as pltpu
from jax.experimental.pallas import tpu_sc as plsc

VOCAB = 100000
HIDDEN = 128
B, L = 1024, 200
EPS = 1e-12

NC, NS, LANES = 2, 16, 16
NW = NC * NS
SEQ_PER_W = B // NW
CHUNK = 100
NV = HIDDEN // LANES


def _sc_body(tokens_hbm, w_hbm, p_hbm, g_hbm, bb_hbm, out_hbm,
             idx_v, rows_v, p_v, g_v, b_v, sem):
    wid = lax.axis_index("s") * NC + lax.axis_index("c")

    pltpu.sync_copy(p_hbm.at[pl.ds(0, L)], p_v)
    pltpu.sync_copy(g_hbm, g_v)
    pltpu.sync_copy(bb_hbm, b_v)

    def seq_body(s, carry):
        seq = wid * SEQ_PER_W + s
        pltpu.sync_copy(tokens_hbm.at[seq], idx_v)
        for half in range(2):
            pltpu.async_copy(w_hbm.at[idx_v.at[half]], rows_v, sem).wait()
            poff = half * CHUNK

            def row_body(r, c2):
                xs = []
                for e in range(NV):
                    x = (rows_v[r, pl.ds(e * LANES, LANES)]
                         + p_v[poff + r, pl.ds(e * LANES, LANES)])
                    xs.append(x)
                sv = xs[0]
                qv = xs[0] * xs[0]
                for e in range(1, NV):
                    sv = sv + xs[e]
                    qv = qv + xs[e] * xs[e]
                mu = jnp.sum(sv) * (1.0 / HIDDEN)
                var = jnp.sum(qv) * (1.0 / HIDDEN) - mu * mu + EPS
                vv = jnp.broadcast_to(var, (LANES,))
                bits = plsc.bitcast(vv, jnp.int32)
                seed = jnp.full((LANES,), 0x5F3759DF, jnp.int32) - (bits >> 1)
                y = plsc.bitcast(seed, jnp.float32)
                for _ in range(3):
                    y = y * (1.5 - 0.5 * vv * y * y)
                mu_v = jnp.broadcast_to(mu, (LANES,))
                for e in range(NV):
                    t = (xs[e] - mu_v) * y
                    rows_v[r, pl.ds(e * LANES, LANES)] = (
                        t * g_v[pl.ds(e * LANES, LANES)]
                        + b_v[pl.ds(e * LANES, LANES)])
                return c2

            lax.fori_loop(0, CHUNK, row_body, 0)
            pltpu.sync_copy(rows_v, out_hbm.at[seq * 2 + half])
        return carry

    lax.fori_loop(0, SEQ_PER_W, seq_body, 0)


@jax.jit
def _sc_embed(tokens3, W, P, gamma, beta):
    mesh = plsc.VectorSubcoreMesh(core_axis_name="c", subcore_axis_name="s")
    f = functools.partial(
        pl.kernel,
        mesh=mesh,
        out_type=jax.ShapeDtypeStruct((B * L // CHUNK, CHUNK, HIDDEN),
                                      jnp.float32),
        scratch_types=[
            pltpu.VMEM((2, CHUNK), jnp.int32),
            pltpu.VMEM((CHUNK, HIDDEN), jnp.float32),
            pltpu.VMEM((L, HIDDEN), jnp.float32),
            pltpu.VMEM((HIDDEN,), jnp.float32),
            pltpu.VMEM((HIDDEN,), jnp.float32),
            pltpu.SemaphoreType.DMA,
        ],
    )(_sc_body)
    return f(tokens3, W, P, gamma, beta)


def kernel(txt_tokens, W, P, gamma, beta):
    tokens3 = txt_tokens.reshape(B, 2, CHUNK)
    out = _sc_embed(tokens3, W, P, gamma, beta)
    embeddings = out.reshape(B, L, HIDDEN)
    position_embeddings = lax.slice(P, (0, 0), (L, HIDDEN))[None]
    return (embeddings, position_embeddings)


# speedup vs baseline: 1.2778x; 1.2778x over previous
import functools

import jax
import jax.numpy as jnp
from jax import lax
from jax.experimental import pallas as pl
from jax.experimental.pallas import tpu as pltpu
from jax.experimental.pallas import tpu_sc as plsc

VOCAB = 100000
HIDDEN = 128
B, L = 1024, 200
EPS = 1e-12

NC, NS, LANES = 2, 16, 16
NW = NC * NS
SEQ_PER_W = B // NW
CHUNK = 100
NV = HIDDEN // LANES

_GATHER_DNUMS = lax.GatherDimensionNumbers(
    offset_dims=(), collapsed_slice_dims=(0,), start_index_map=(0,))


def _lane_shuffle(v, perm):
    return lax.gather(v, perm[:, None], _GATHER_DNUMS, (1,),
                      mode=lax.GatherScatterMode.PROMISE_IN_BOUNDS)


def _sc_body(tokens_hbm, w_hbm, p_hbm, g_hbm, bb_hbm, out_hbm,
             idx_v, rows_v, p_v, g_v, b_v, sem):
    wid = lax.axis_index("s") * NC + lax.axis_index("c")

    pltpu.sync_copy(p_hbm.at[pl.ds(0, L)], p_v)
    pltpu.sync_copy(g_hbm, g_v)
    pltpu.sync_copy(bb_hbm, b_v)

    def seq_body(s, carry):
        seq = wid * SEQ_PER_W + s
        pltpu.sync_copy(tokens_hbm.at[seq], idx_v)
        for half in range(2):
            pltpu.async_copy(w_hbm.at[idx_v.at[half]], rows_v, sem).wait()
            poff = half * CHUNK

            def row_body(r, c2):
                lane = lax.iota(jnp.int32, LANES)
                xs = []
                for e in range(NV):
                    x = (rows_v[r, pl.ds(e * LANES, LANES)]
                         + p_v[poff + r, pl.ds(e * LANES, LANES)])
                    xs.append(x)
                sv = xs[0]
                qv = xs[0] * xs[0]
                for e in range(1, NV):
                    sv = sv + xs[e]
                    qv = qv + xs[e] * xs[e]
                for sh in (8, 4, 2, 1):
                    perm = lane ^ sh
                    sv = sv + _lane_shuffle(sv, perm)
                    qv = qv + _lane_shuffle(qv, perm)
                mu_v = sv * (1.0 / HIDDEN)
                vv = qv * (1.0 / HIDDEN) - mu_v * mu_v + EPS
                bits = lax.bitcast_convert_type(vv, jnp.int32)
                seed = jnp.full((LANES,), 0x5F3759DF, jnp.int32) - (bits >> 1)
                y = lax.bitcast_convert_type(seed, jnp.float32)
                for _ in range(3):
                    y = y * (1.5 - 0.5 * vv * y * y)
                for e in range(NV):
                    t = (xs[e] - mu_v) * y
                    rows_v[r, pl.ds(e * LANES, LANES)] = (
                        t * g_v[pl.ds(e * LANES, LANES)]
                        + b_v[pl.ds(e * LANES, LANES)])
                return c2

            lax.fori_loop(0, CHUNK, row_body, 0)
            pltpu.sync_copy(rows_v, out_hbm.at[seq * 2 + half])
        return carry

    lax.fori_loop(0, SEQ_PER_W, seq_body, 0)


@jax.jit
def _sc_embed(tokens3, W, P, gamma, beta):
    mesh = plsc.VectorSubcoreMesh(core_axis_name="c", subcore_axis_name="s")
    f = functools.partial(
        pl.kernel,
        mesh=mesh,
        out_type=jax.ShapeDtypeStruct((B * L // CHUNK, CHUNK, HIDDEN),
                                      jnp.float32),
        scratch_types=[
            pltpu.VMEM((2, CHUNK), jnp.int32),
            pltpu.VMEM((CHUNK, HIDDEN), jnp.float32),
            pltpu.VMEM((L, HIDDEN), jnp.float32),
            pltpu.VMEM((HIDDEN,), jnp.float32),
            pltpu.VMEM((HIDDEN,), jnp.float32),
            pltpu.SemaphoreType.DMA,
        ],
    )(_sc_body)
    return f(tokens3, W, P, gamma, beta)


def kernel(txt_tokens, W, P, gamma, beta):
    tokens3 = txt_tokens.reshape(B, 2, CHUNK)
    out = _sc_embed(tokens3, W, P, gamma, beta)
    embeddings = out.reshape(B, L, HIDDEN)
    position_embeddings = lax.slice(P, (0, 0), (L, HIDDEN))[None]
    return (embeddings, position_embeddings)


# speedup vs baseline: 3.3325x; 2.6079x over previous
import functools

import jax
import jax.numpy as jnp
from jax import lax
from jax.experimental import pallas as pl
from jax.experimental.pallas import tpu as pltpu
from jax.experimental.pallas import tpu_sc as plsc

VOCAB = 100000
HIDDEN = 128
B, L = 1024, 200
EPS = 1e-12

NC, NS, LANES = 2, 16, 16
NW = NC * NS
SEQ_PER_W = B // NW
CHUNK = 100
NV = HIDDEN // LANES
NBUF = 4
NCHUNK = SEQ_PER_W * 2
NI = NCHUNK // NBUF

_GATHER_DNUMS = lax.GatherDimensionNumbers(
    offset_dims=(), collapsed_slice_dims=(0,), start_index_map=(0,))


def _lane_shuffle(v, perm):
    return lax.gather(v, perm[:, None], _GATHER_DNUMS, (1,),
                      mode=lax.GatherScatterMode.PROMISE_IN_BOUNDS)


def _ln_rows(rv, p_v, poff, gsbs):
    def row_body(r2, gb):
        gs, bs = gb[:NV], gb[NV:]
        lane = lax.iota(jnp.int32, LANES)
        for k in range(2):
            r = r2 * 2 + k
            xs = []
            for e in range(NV):
                x = (rv[r, pl.ds(e * LANES, LANES)]
                     + p_v[poff + r, pl.ds(e * LANES, LANES)])
                xs.append(x)
            sv = xs[0]
            qv = xs[0] * xs[0]
            for e in range(1, NV):
                sv = sv + xs[e]
                qv = qv + xs[e] * xs[e]
            for sh in (8, 4, 2, 1):
                perm = lane ^ sh
                sv = sv + _lane_shuffle(sv, perm)
                qv = qv + _lane_shuffle(qv, perm)
            mu_v = sv * (1.0 / HIDDEN)
            vv = qv * (1.0 / HIDDEN) - mu_v * mu_v + EPS
            bits = lax.bitcast_convert_type(vv, jnp.int32)
            seed = jnp.full((LANES,), 0x5F3759DF, jnp.int32) - (bits >> 1)
            y = lax.bitcast_convert_type(seed, jnp.float32)
            hv = 0.5 * vv
            for _ in range(2):
                y = y * (1.5 - hv * y * y)
            for e in range(NV):
                rv[r, pl.ds(e * LANES, LANES)] = (
                    (xs[e] - mu_v) * y * gs[e] + bs[e])
        return gb

    return lax.fori_loop(0, CHUNK // 2, row_body, gsbs)


def _sc_body(tokens_hbm, w_hbm, p_hbm, g_hbm, bb_hbm, out_hbm,
             idx_all, rows, p_v, gb_v, sem_g, sem_o):
    wid = lax.axis_index("s") * NC + lax.axis_index("c")
    out_base = wid * NCHUNK

    pltpu.sync_copy(tokens_hbm.at[pl.ds(wid * SEQ_PER_W, SEQ_PER_W)], idx_all)
    pltpu.sync_copy(p_hbm.at[pl.ds(0, L)], p_v)
    pltpu.sync_copy(g_hbm, gb_v.at[0])
    pltpu.sync_copy(bb_hbm, gb_v.at[1])

    def gather(seq_local, half, buf):
        return pltpu.make_async_copy(
            w_hbm.at[idx_all.at[seq_local, half]], rows.at[buf],
            sem_g.at[buf])

    def out_copy(chunk, buf):
        return pltpu.make_async_copy(
            rows.at[buf], out_hbm.at[out_base + chunk], sem_o.at[buf])

    gather(0, 0, 0).start()
    gather(0, 1, 1).start()

    gs = tuple(gb_v[0, pl.ds(e * LANES, LANES)] for e in range(NV))
    bs = tuple(gb_v[1, pl.ds(e * LANES, LANES)] for e in range(NV))

    def outer(i, gsbs):
        for b in range(NBUF):
            c = NBUF * i + b
            sl = 2 * i + b // 2
            hh = b % 2
            gather(sl, hh, b).wait()
            nb = (b + 2) % NBUF
            sl_n = 2 * i + (b + 2) // 2
            if b < 2:
                @pl.when(i >= 1)
                def _wait_old():
                    out_copy(c - 2, nb).wait()
                gather(sl_n, hh, nb).start()
            else:
                @pl.when(i < NI - 1)
                def _wait_and_issue():
                    out_copy(c - 2, nb).wait()
                    gather(sl_n, hh, nb).start()
            gsbs = _ln_rows(rows.at[b], p_v, hh * CHUNK, gsbs)
            out_copy(c, b).start()
        return gsbs

    lax.fori_loop(0, NI, outer, gs + bs)

    for b in range(NBUF):
        out_copy(NCHUNK - NBUF + b, b).wait()


@jax.jit
def _sc_embed(tokens3, W, P, gamma, beta):
    mesh = plsc.VectorSubcoreMesh(core_axis_name="c", subcore_axis_name="s")
    f = functools.partial(
        pl.kernel,
        mesh=mesh,
        out_type=jax.ShapeDtypeStruct((B * L // CHUNK, CHUNK, HIDDEN),
                                      jnp.float32),
        scratch_types=[
            pltpu.VMEM((SEQ_PER_W, 2, CHUNK), jnp.int32),
            pltpu.VMEM((NBUF, CHUNK, HIDDEN), jnp.float32),
            pltpu.VMEM((L, HIDDEN), jnp.float32),
            pltpu.VMEM((2, HIDDEN), jnp.float32),
            pltpu.SemaphoreType.DMA((NBUF,)),
            pltpu.SemaphoreType.DMA((NBUF,)),
        ],
    )(_sc_body)
    return f(tokens3, W, P, gamma, beta)


def kernel(txt_tokens, W, P, gamma, beta):
    tokens3 = txt_tokens.reshape(B, 2, CHUNK)
    out = _sc_embed(tokens3, W, P, gamma, beta)
    embeddings = out.reshape(B, L, HIDDEN)
    position_embeddings = lax.slice(P, (0, 0), (L, HIDDEN))[None]
    return (embeddings, position_embeddings)
